# baseline (speedup 1.0000x reference)
import jax
import jax.numpy as jnp
from jax.experimental import pallas as pl
from jax.experimental.pallas import tpu as pltpu

_BS = 1024
_BB = 2


def _fused_ln_kernel(x_ref, p1_ref, p2_ref, g_ref, b_ref, o_ref):
    x = x_ref[...]
    e = p1_ref[...][None] * x + p2_ref[...][None]
    mean = jnp.mean(e, axis=2, keepdims=True)
    m2 = jnp.mean(e * e, axis=2, keepdims=True)
    var = m2 - mean * mean
    scale = jax.lax.rsqrt(var + 1e-12) * g_ref[...]
    o_ref[...] = (e - mean) * scale + b_ref[...]


def kernel(inputs_embeds, pos_table, pos_table2, ln_gamma, ln_beta,
           past_key_values_length):
    del past_key_values_length
    B, S, H = inputs_embeds.shape
    g = ln_gamma.reshape(1, H)
    b = ln_beta.reshape(1, H)
    return pl.pallas_call(
        _fused_ln_kernel,
        grid=(S // _BS, B // _BB),
        in_specs=[
            pl.BlockSpec((_BB, _BS, H), lambda i, j: (j, i, 0)),
            pl.BlockSpec((_BS, H), lambda i, j: (i, 0)),
            pl.BlockSpec((_BS, H), lambda i, j: (i, 0)),
            pl.BlockSpec((1, H), lambda i, j: (0, 0)),
            pl.BlockSpec((1, H), lambda i, j: (0, 0)),
        ],
        out_specs=pl.BlockSpec((_BB, _BS, H), lambda i, j: (j, i, 0)),
        out_shape=jax.ShapeDtypeStruct((B, S, H), inputs_embeds.dtype),
        compiler_params=pltpu.CompilerParams(
            dimension_semantics=("parallel", "parallel")),
    )(inputs_embeds, pos_table, pos_table2, g, b)


# speedup vs baseline: 1.1012x; 1.1012x over previous
import jax
import jax.numpy as jnp
from jax.experimental import pallas as pl
from jax.experimental.pallas import tpu as pltpu

_BS = 512


def _fused_ln_kernel(x_ref, p1_ref, p2_ref, g_ref, b_ref, o_ref):
    x = x_ref[...]
    e = p1_ref[...][None] * x + p2_ref[...][None]
    mean = jnp.mean(e, axis=2, keepdims=True)
    m2 = jnp.mean(e * e, axis=2, keepdims=True)
    var = m2 - mean * mean
    scale = jax.lax.rsqrt(var + 1e-12) * g_ref[...]
    o_ref[...] = (e - mean) * scale + b_ref[...]


def kernel(inputs_embeds, pos_table, pos_table2, ln_gamma, ln_beta,
           past_key_values_length):
    del past_key_values_length
    B, S, H = inputs_embeds.shape
    g = ln_gamma.reshape(1, H)
    b = ln_beta.reshape(1, H)
    return pl.pallas_call(
        _fused_ln_kernel,
        grid=(S // _BS,),
        in_specs=[
            pl.BlockSpec((B, _BS, H), lambda i: (0, i, 0)),
            pl.BlockSpec((_BS, H), lambda i: (i, 0)),
            pl.BlockSpec((_BS, H), lambda i: (i, 0)),
            pl.BlockSpec((1, H), lambda i: (0, 0)),
            pl.BlockSpec((1, H), lambda i: (0, 0)),
        ],
        out_specs=pl.BlockSpec((B, _BS, H), lambda i: (0, i, 0)),
        out_shape=jax.ShapeDtypeStruct((B, S, H), inputs_embeds.dtype),
        compiler_params=pltpu.CompilerParams(
            dimension_semantics=("parallel",)),
    )(inputs_embeds, pos_table, pos_table2, g, b)
